# baseline (speedup 1.0000x reference)
import functools

import jax
import jax.numpy as jnp
from jax import lax
from jax.experimental import pallas as pl
from jax.experimental.pallas import tpu as pltpu
from jax.experimental.pallas import tpu_sc as plsc

B = 8
N = 2048
IB = 256
NIB = N // IB
RADIUS = 0.07
H2 = 0.03 * 0.03
EPS = 1e-12
BIG = 1e30
ALPHA = 1.0

NTEC = 32
QPT = (B * N) // NTEC
NG = QPT // 16


def _chamfer_body(pred_blk, gt_t, out_ref, colmin, accs):
    b = pl.program_id(0)
    ib = pl.program_id(1)

    @pl.when((b == 0) & (ib == 0))
    def _init():
        accs[0] = 0.0
        accs[1] = 0.0

    pi = pred_blk[0]
    pxi = pi[:, 0:1]
    pyi = pi[:, 1:2]
    pzi = pi[:, 2:3]
    a2 = (pxi * pxi + pyi * pyi) + pzi * pzi

    gt = gt_t[0]
    gx = gt[0:1, :]
    gy = gt[1:2, :]
    gz = gt[2:3, :]
    g2 = (gx * gx + gy * gy) + gz * gz

    ab_g = jax.lax.dot_general(
        pi.astype(jnp.bfloat16), gt.astype(jnp.bfloat16),
        (((1,), (0,)), ((), ())),
        preferred_element_type=jnp.float32)
    d_pg = jnp.maximum((a2 + g2) - 2.0 * ab_g, 0.0)

    accs[0] = accs[0] + jnp.sum(jnp.min(d_pg, axis=1))

    col = jnp.min(d_pg, axis=0, keepdims=True)

    @pl.when(ib == 0)
    def _c0():
        colmin[...] = col

    @pl.when(ib > 0)
    def _c1():
        colmin[...] = jnp.minimum(colmin[...], col)

    @pl.when(ib == NIB - 1)
    def _cfin():
        accs[1] = accs[1] + jnp.sum(colmin[...])

    @pl.when((b == B - 1) & (ib == NIB - 1))
    def _out():
        out_ref[0] = accs[0]
        out_ref[1] = accs[1]


def _rep_body(pe_hbm, pb_hbm, out_hbm,
              xe, ye, ze, xb, yb, zb, p2v, accv):
    c = lax.axis_index("c")
    s = lax.axis_index("s")
    wid = s * 2 + c
    b = wid // 4
    q = wid % 4

    base = b * 3 * N
    pltpu.sync_copy(pe_hbm.at[pl.ds(base, N)], xe)
    pltpu.sync_copy(pe_hbm.at[pl.ds(base + N, N)], ye)
    pltpu.sync_copy(pe_hbm.at[pl.ds(base + 2 * N, N)], ze)
    pltpu.sync_copy(pb_hbm.at[pl.ds(base, N)], xb)
    pltpu.sync_copy(pb_hbm.at[pl.ds(base + N, N)], yb)
    pltpu.sync_copy(pb_hbm.at[pl.ds(base + 2 * N, N)], zb)

    def p2_step(k, carry):
        xv = xe[pl.ds(k * 16, 16)]
        yv = ye[pl.ds(k * 16, 16)]
        zv = ze[pl.ds(k * 16, 16)]
        p2v[pl.ds(k * 16, 16)] = (xv * xv + yv * yv) + zv * zv
        return carry

    lax.fori_loop(0, N // 16, p2_step, jnp.int32(0))

    zero = jnp.zeros((16,), jnp.float32)
    big = jnp.full((16,), BIG, jnp.float32)

    def group_step(g, acc):
        i0 = q * QPT + g * 16
        xi = xe[pl.ds(i0, 16)]
        yi = ye[pl.ds(i0, 16)]
        zi = ze[pl.ds(i0, 16)]
        xib = xb[pl.ds(i0, 16)]
        yib = yb[pl.ds(i0, 16)]
        zib = zb[pl.ds(i0, 16)]
        a2i = p2v[pl.ds(i0, 16)]

        def chunk_step(ch, st):
            m1, m2, m3, m4, m5, e1, e2, e3, e4, e5 = st
            xjv = xb[pl.ds(ch * 16, 16)]
            yjv = yb[pl.ds(ch * 16, 16)]
            zjv = zb[pl.ds(ch * 16, 16)]
            pjv = p2v[pl.ds(ch * 16, 16)]
            xev = xe[pl.ds(ch * 16, 16)]
            yev = ye[pl.ds(ch * 16, 16)]
            zev = ze[pl.ds(ch * 16, 16)]
            for k in range(16):
                dot = (xib * xjv[k] + yib * yjv[k]) + zib * zjv[k]
                v = jnp.maximum((a2i + pjv[k]) - 2.0 * dot, 0.0)
                dx = xi - xev[k]
                dy = yi - yev[k]
                dz = zi - zev[k]
                jf = (dx * dx + dy * dy) + dz * dz
                c1 = v < m1
                c2 = v < m2
                c3 = v < m3
                c4 = v < m4
                c5 = v < m5
                nm1 = jnp.where(c1, v, m1)
                nm2 = jnp.where(c1, m1, jnp.where(c2, v, m2))
                nm3 = jnp.where(c2, m2, jnp.where(c3, v, m3))
                nm4 = jnp.where(c3, m3, jnp.where(c4, v, m4))
                nm5 = jnp.where(c4, m4, jnp.where(c5, v, m5))
                ne1 = jnp.where(c1, jf, e1)
                ne2 = jnp.where(c1, e1, jnp.where(c2, jf, e2))
                ne3 = jnp.where(c2, e2, jnp.where(c3, jf, e3))
                ne4 = jnp.where(c3, e3, jnp.where(c4, jf, e4))
                ne5 = jnp.where(c4, e4, jnp.where(c5, jf, e5))
                m1, m2, m3, m4, m5 = nm1, nm2, nm3, nm4, nm5
                e1, e2, e3, e4, e5 = ne1, ne2, ne3, ne4, ne5
            return (m1, m2, m3, m4, m5, e1, e2, e3, e4, e5)

        st = lax.fori_loop(0, N // 16, chunk_step,
                           (big, big, big, big, big,
                            zero, zero, zero, zero, zero))

        contrib = zero
        for ek in st[6:10]:
            d2 = jnp.maximum(ek, EPS)
            ii = lax.bitcast_convert_type(d2, jnp.int32)
            ii = jnp.int32(0x5F3759DF) - lax.shift_right_arithmetic(
                ii, jnp.int32(1))
            y = lax.bitcast_convert_type(ii, jnp.float32)
            y = y * (1.5 - 0.5 * d2 * y * y)
            y = y * (1.5 - 0.5 * d2 * y * y)
            y = y * (1.5 - 0.5 * d2 * y * y)
            dist = d2 * y
            w = jnp.exp(d2 * (-1.0 / H2))
            contrib = contrib + (RADIUS - dist) * w
        return acc + contrib

    acc = lax.fori_loop(0, NG, group_step, zero)
    accv[...] = acc
    pltpu.sync_copy(accv, out_hbm.at[pl.ds(wid * 16, 16)])


def kernel(pred, gt, pcd_radius):
    del pcd_radius
    pred_t = jnp.swapaxes(pred, 1, 2)
    gt_t = jnp.swapaxes(gt, 1, 2)
    pred_tb = lax.optimization_barrier(
        pred_t.astype(jnp.bfloat16)).astype(jnp.float32)

    cd = pl.pallas_call(
        _chamfer_body,
        grid=(B, NIB),
        in_specs=[
            pl.BlockSpec((1, IB, 3), lambda b, i: (b, i, 0)),
            pl.BlockSpec((1, 3, N), lambda b, i: (b, 0, 0)),
        ],
        out_specs=pl.BlockSpec(memory_space=pltpu.SMEM),
        out_shape=jax.ShapeDtypeStruct((2,), jnp.float32),
        scratch_shapes=[
            pltpu.VMEM((1, N), jnp.float32),
            pltpu.SMEM((2,), jnp.float32),
        ],
    )(pred, gt_t)

    rep = pl.kernel(
        _rep_body,
        out_type=jax.ShapeDtypeStruct((NTEC * 16,), jnp.float32),
        mesh=plsc.VectorSubcoreMesh(core_axis_name="c", subcore_axis_name="s"),
        scratch_types=[
            pltpu.VMEM((N,), jnp.float32),
            pltpu.VMEM((N,), jnp.float32),
            pltpu.VMEM((N,), jnp.float32),
            pltpu.VMEM((N,), jnp.float32),
            pltpu.VMEM((N,), jnp.float32),
            pltpu.VMEM((N,), jnp.float32),
            pltpu.VMEM((N,), jnp.float32),
            pltpu.VMEM((16,), jnp.float32),
        ],
    )(pred_t.reshape(-1), pred_tb.reshape(-1))

    cd_loss = (cd[0] + cd[1]) / jnp.float32(B * N) * 100.0
    uniform = jnp.sum(rep) / jnp.float32(B * N * 4)
    return (cd_loss, ALPHA * uniform)


# speedup vs baseline: 1.0250x; 1.0250x over previous
import functools

import jax
import jax.numpy as jnp
from jax import lax
from jax.experimental import pallas as pl
from jax.experimental.pallas import tpu as pltpu
from jax.experimental.pallas import tpu_sc as plsc

B = 8
N = 2048
IB = 256
NIB = N // IB
RADIUS = 0.07
H2 = 0.03 * 0.03
EPS = 1e-12
BIG = 1e30
ALPHA = 1.0

NTEC = 32
QPT = (B * N) // NTEC
NG = QPT // 16


def _chamfer_body(pred_blk, gt_t, out_ref, colmin, accs):
    b = pl.program_id(0)
    ib = pl.program_id(1)

    @pl.when((b == 0) & (ib == 0))
    def _init():
        accs[0] = 0.0
        accs[1] = 0.0

    pi = pred_blk[0]
    pxi = pi[:, 0:1]
    pyi = pi[:, 1:2]
    pzi = pi[:, 2:3]
    a2 = (pxi * pxi + pyi * pyi) + pzi * pzi

    gt = gt_t[0]
    gx = gt[0:1, :]
    gy = gt[1:2, :]
    gz = gt[2:3, :]
    g2 = (gx * gx + gy * gy) + gz * gz

    ab_g = jax.lax.dot_general(
        pi.astype(jnp.bfloat16), gt.astype(jnp.bfloat16),
        (((1,), (0,)), ((), ())),
        preferred_element_type=jnp.float32)
    d_pg = jnp.maximum((a2 + g2) - 2.0 * ab_g, 0.0)

    accs[0] = accs[0] + jnp.sum(jnp.min(d_pg, axis=1))

    col = jnp.min(d_pg, axis=0, keepdims=True)

    @pl.when(ib == 0)
    def _c0():
        colmin[...] = col

    @pl.when(ib > 0)
    def _c1():
        colmin[...] = jnp.minimum(colmin[...], col)

    @pl.when(ib == NIB - 1)
    def _cfin():
        accs[1] = accs[1] + jnp.sum(colmin[...])

    @pl.when((b == B - 1) & (ib == NIB - 1))
    def _out():
        out_ref[0] = accs[0]
        out_ref[1] = accs[1]


def _rep_body(pe_hbm, pb_hbm, out_hbm,
              xe, ye, ze, xb, yb, zb, p2v, accv):
    c = lax.axis_index("c")
    s = lax.axis_index("s")
    wid = s * 2 + c
    b = wid // 4
    q = wid % 4

    base = b * 3 * N
    pltpu.sync_copy(pe_hbm.at[pl.ds(base, N)], xe)
    pltpu.sync_copy(pe_hbm.at[pl.ds(base + N, N)], ye)
    pltpu.sync_copy(pe_hbm.at[pl.ds(base + 2 * N, N)], ze)
    pltpu.sync_copy(pb_hbm.at[pl.ds(base, N)], xb)
    pltpu.sync_copy(pb_hbm.at[pl.ds(base + N, N)], yb)
    pltpu.sync_copy(pb_hbm.at[pl.ds(base + 2 * N, N)], zb)

    def p2_step(k, carry):
        xv = xe[pl.ds(k * 16, 16)]
        yv = ye[pl.ds(k * 16, 16)]
        zv = ze[pl.ds(k * 16, 16)]
        p2v[pl.ds(k * 16, 16)] = (xv * xv + yv * yv) + zv * zv
        return carry

    lax.fori_loop(0, N // 16, p2_step, jnp.int32(0))

    zero = jnp.zeros((16,), jnp.float32)
    big = jnp.full((16,), BIG, jnp.float32)

    def _ins(st, v, e):
        m1, m2, m3, m4, m5, e1, e2, e3, e4, e5 = st
        c1 = v < m1
        c2 = v < m2
        c3 = v < m3
        c4 = v < m4
        c5 = v < m5
        return (jnp.where(c1, v, m1),
                jnp.where(c1, m1, jnp.where(c2, v, m2)),
                jnp.where(c2, m2, jnp.where(c3, v, m3)),
                jnp.where(c3, m3, jnp.where(c4, v, m4)),
                jnp.where(c4, m4, jnp.where(c5, v, m5)),
                jnp.where(c1, e, e1),
                jnp.where(c1, e1, jnp.where(c2, e, e2)),
                jnp.where(c2, e2, jnp.where(c3, e, e3)),
                jnp.where(c3, e3, jnp.where(c4, e, e4)),
                jnp.where(c4, e4, jnp.where(c5, e, e5)))

    NSTR = 4
    CPS = N // 16 // NSTR

    def group_step(g, acc):
        i0 = q * QPT + g * 16
        xi = xe[pl.ds(i0, 16)]
        yi = ye[pl.ds(i0, 16)]
        zi = ze[pl.ds(i0, 16)]
        xib = xb[pl.ds(i0, 16)]
        yib = yb[pl.ds(i0, 16)]
        zib = zb[pl.ds(i0, 16)]
        a2i = p2v[pl.ds(i0, 16)]

        def chunk_step(ch, sts):
            out = []
            for s in range(NSTR):
                st = sts[s]
                j0 = (s * CPS + ch) * 16
                xjv = xb[pl.ds(j0, 16)]
                yjv = yb[pl.ds(j0, 16)]
                zjv = zb[pl.ds(j0, 16)]
                pjv = p2v[pl.ds(j0, 16)]
                xev = xe[pl.ds(j0, 16)]
                yev = ye[pl.ds(j0, 16)]
                zev = ze[pl.ds(j0, 16)]
                for k in range(16):
                    dot = (xib * xjv[k] + yib * yjv[k]) + zib * zjv[k]
                    v = jnp.maximum((a2i + pjv[k]) - 2.0 * dot, 0.0)
                    dx = xi - xev[k]
                    dy = yi - yev[k]
                    dz = zi - zev[k]
                    de = (dx * dx + dy * dy) + dz * dz
                    st = _ins(st, v, de)
                out.append(st)
            return tuple(out)

        init = (big, big, big, big, big, zero, zero, zero, zero, zero)
        sts = lax.fori_loop(0, CPS, chunk_step, (init,) * NSTR)

        merged = sts[0]
        for s in range(1, NSTR):
            for slot in range(5):
                merged = _ins(merged, sts[s][slot], sts[s][5 + slot])

        contrib = zero
        for ek in merged[6:10]:
            d2 = jnp.maximum(ek, EPS)
            ii = lax.bitcast_convert_type(d2, jnp.int32)
            ii = jnp.int32(0x5F3759DF) - lax.shift_right_arithmetic(
                ii, jnp.int32(1))
            y = lax.bitcast_convert_type(ii, jnp.float32)
            y = y * (1.5 - 0.5 * d2 * y * y)
            y = y * (1.5 - 0.5 * d2 * y * y)
            y = y * (1.5 - 0.5 * d2 * y * y)
            dist = d2 * y
            w = jnp.exp(d2 * (-1.0 / H2))
            contrib = contrib + (RADIUS - dist) * w
        return acc + contrib

    acc = lax.fori_loop(0, NG, group_step, zero)
    accv[...] = acc
    pltpu.sync_copy(accv, out_hbm.at[pl.ds(wid * 16, 16)])


def kernel(pred, gt, pcd_radius):
    del pcd_radius
    pred_t = jnp.swapaxes(pred, 1, 2)
    gt_t = jnp.swapaxes(gt, 1, 2)
    pred_tb = lax.optimization_barrier(
        pred_t.astype(jnp.bfloat16)).astype(jnp.float32)

    cd = pl.pallas_call(
        _chamfer_body,
        grid=(B, NIB),
        in_specs=[
            pl.BlockSpec((1, IB, 3), lambda b, i: (b, i, 0)),
            pl.BlockSpec((1, 3, N), lambda b, i: (b, 0, 0)),
        ],
        out_specs=pl.BlockSpec(memory_space=pltpu.SMEM),
        out_shape=jax.ShapeDtypeStruct((2,), jnp.float32),
        scratch_shapes=[
            pltpu.VMEM((1, N), jnp.float32),
            pltpu.SMEM((2,), jnp.float32),
        ],
    )(pred, gt_t)

    rep = pl.kernel(
        _rep_body,
        out_type=jax.ShapeDtypeStruct((NTEC * 16,), jnp.float32),
        mesh=plsc.VectorSubcoreMesh(core_axis_name="c", subcore_axis_name="s"),
        scratch_types=[
            pltpu.VMEM((N,), jnp.float32),
            pltpu.VMEM((N,), jnp.float32),
            pltpu.VMEM((N,), jnp.float32),
            pltpu.VMEM((N,), jnp.float32),
            pltpu.VMEM((N,), jnp.float32),
            pltpu.VMEM((N,), jnp.float32),
            pltpu.VMEM((N,), jnp.float32),
            pltpu.VMEM((16,), jnp.float32),
        ],
    )(pred_t.reshape(-1), pred_tb.reshape(-1))

    cd_loss = (cd[0] + cd[1]) / jnp.float32(B * N) * 100.0
    uniform = jnp.sum(rep) / jnp.float32(B * N * 4)
    return (cd_loss, ALPHA * uniform)


# speedup vs baseline: 2.0428x; 1.9929x over previous
import functools

import jax
import jax.numpy as jnp
from jax import lax
from jax.experimental import pallas as pl
from jax.experimental.pallas import tpu as pltpu
from jax.experimental.pallas import tpu_sc as plsc

B = 8
N = 2048
IB = 256
NIB = N // IB
RADIUS = 0.07
H2 = 0.03 * 0.03
EPS = 1e-12
BIG = 1e30
ALPHA = 1.0

BTC = 5
KB = B - BTC
NTEC = 32
QPT = (KB * N) // NTEC
NG = QPT // 16
NSTR = 4
CPS = N // 16 // NSTR


def _tc_body(pred_blk, pred_t, gt_t, out_ref, colmin, accs):
    b = pl.program_id(0)
    ib = pl.program_id(1)

    @pl.when((b == 0) & (ib == 0))
    def _init():
        accs[0] = 0.0
        accs[1] = 0.0
        accs[2] = 0.0

    pi = pred_blk[0]
    pxi = pi[:, 0:1]
    pyi = pi[:, 1:2]
    pzi = pi[:, 2:3]
    a2 = (pxi * pxi + pyi * pyi) + pzi * pzi

    gt = gt_t[0]
    gx = gt[0:1, :]
    gy = gt[1:2, :]
    gz = gt[2:3, :]
    g2 = (gx * gx + gy * gy) + gz * gz

    ab_g = jax.lax.dot_general(
        pi.astype(jnp.bfloat16), gt.astype(jnp.bfloat16),
        (((1,), (0,)), ((), ())),
        preferred_element_type=jnp.float32)
    d_pg = jnp.maximum((a2 + g2) - 2.0 * ab_g, 0.0)

    accs[0] = accs[0] + jnp.sum(jnp.min(d_pg, axis=1))

    col = jnp.min(d_pg, axis=0, keepdims=True)

    @pl.when(ib == 0)
    def _c0():
        colmin[...] = col

    @pl.when(ib > 0)
    def _c1():
        colmin[...] = jnp.minimum(colmin[...], col)

    @pl.when(ib == NIB - 1)
    def _cfin():
        accs[1] = accs[1] + jnp.sum(colmin[...])

    @pl.when(b < BTC)
    def _repulsion():
        pt = pred_t[0]
        px = pt[0:1, :]
        py = pt[1:2, :]
        pz = pt[2:3, :]
        p2 = (px * px + py * py) + pz * pz

        ab_p = jax.lax.dot_general(
            pi.astype(jnp.bfloat16), pt.astype(jnp.bfloat16),
            (((1,), (0,)), ((), ())),
            preferred_element_type=jnp.float32)
        v = jnp.maximum((a2 + p2) - 2.0 * ab_p, 0.0)

        dxp = pxi - px
        dyp = pyi - py
        dzp = pzi - pz
        dex = dxp * dxp + dyp * dyp + dzp * dzp
        dist2 = jnp.maximum(dex, EPS)
        cont = (RADIUS - jnp.sqrt(dist2)) * jnp.exp(dist2 * (-1.0 / H2))

        jvec = jax.lax.broadcasted_iota(jnp.int32, (IB, N), 1)
        rep = jnp.float32(0.0)
        for r in range(5):
            m = jnp.min(v, axis=1, keepdims=True)
            eq = v == m
            jm = jnp.min(jnp.where(eq, jvec, jnp.int32(N)), axis=1,
                         keepdims=True)
            sel = jvec == jm
            if r > 0:
                rep = rep + jnp.sum(jnp.where(sel, cont, 0.0))
            v = jnp.where(sel, BIG, v)
        accs[2] = accs[2] + rep

    @pl.when((b == B - 1) & (ib == NIB - 1))
    def _out():
        out_ref[0] = accs[0]
        out_ref[1] = accs[1]
        out_ref[2] = accs[2]


def _rep_body(pe_hbm, pb_hbm, out_hbm,
              xe, ye, ze, xb, yb, zb, p2v, accv):
    c = lax.axis_index("c")
    s = lax.axis_index("s")
    wid = s * 2 + c

    for kb in range(KB):
        hb = (BTC + kb) * 3 * N
        lb = kb * N
        pltpu.sync_copy(pe_hbm.at[pl.ds(hb, N)], xe.at[pl.ds(lb, N)])
        pltpu.sync_copy(pe_hbm.at[pl.ds(hb + N, N)], ye.at[pl.ds(lb, N)])
        pltpu.sync_copy(pe_hbm.at[pl.ds(hb + 2 * N, N)], ze.at[pl.ds(lb, N)])
        pltpu.sync_copy(pb_hbm.at[pl.ds(hb, N)], xb.at[pl.ds(lb, N)])
        pltpu.sync_copy(pb_hbm.at[pl.ds(hb + N, N)], yb.at[pl.ds(lb, N)])
        pltpu.sync_copy(pb_hbm.at[pl.ds(hb + 2 * N, N)], zb.at[pl.ds(lb, N)])

    def p2_step(k, carry):
        xv = xe[pl.ds(k * 16, 16)]
        yv = ye[pl.ds(k * 16, 16)]
        zv = ze[pl.ds(k * 16, 16)]
        p2v[pl.ds(k * 16, 16)] = (xv * xv + yv * yv) + zv * zv
        return carry

    lax.fori_loop(0, KB * N // 16, p2_step, jnp.int32(0))

    zero = jnp.zeros((16,), jnp.float32)
    big = jnp.full((16,), BIG, jnp.float32)

    def _ins(st, v, e):
        m1, m2, m3, m4, m5, e1, e2, e3, e4, e5 = st
        c1 = v < m1
        c2 = v < m2
        c3 = v < m3
        c4 = v < m4
        c5 = v < m5
        return (jnp.where(c1, v, m1),
                jnp.where(c1, m1, jnp.where(c2, v, m2)),
                jnp.where(c2, m2, jnp.where(c3, v, m3)),
                jnp.where(c3, m3, jnp.where(c4, v, m4)),
                jnp.where(c4, m4, jnp.where(c5, v, m5)),
                jnp.where(c1, e, e1),
                jnp.where(c1, e1, jnp.where(c2, e, e2)),
                jnp.where(c2, e2, jnp.where(c3, e, e3)),
                jnp.where(c3, e3, jnp.where(c4, e, e4)),
                jnp.where(c4, e4, jnp.where(c5, e, e5)))

    def group_step(g, acc):
        qb = wid * QPT + g * 16
        bi = qb // N
        i0 = bi * N + (qb - bi * N)
        xi = xe[pl.ds(i0, 16)]
        yi = ye[pl.ds(i0, 16)]
        zi = ze[pl.ds(i0, 16)]
        xib = xb[pl.ds(i0, 16)]
        yib = yb[pl.ds(i0, 16)]
        zib = zb[pl.ds(i0, 16)]
        a2i = p2v[pl.ds(i0, 16)]
        jb0 = bi * N

        def chunk_step(ch, sts):
            out = []
            for st_i in range(NSTR):
                st = sts[st_i]
                j0 = jb0 + (st_i * CPS + ch) * 16
                xjv = xb[pl.ds(j0, 16)]
                yjv = yb[pl.ds(j0, 16)]
                zjv = zb[pl.ds(j0, 16)]
                pjv = p2v[pl.ds(j0, 16)]
                xev = xe[pl.ds(j0, 16)]
                yev = ye[pl.ds(j0, 16)]
                zev = ze[pl.ds(j0, 16)]
                for k in range(16):
                    dot = (xib * xjv[k] + yib * yjv[k]) + zib * zjv[k]
                    v = jnp.maximum((a2i + pjv[k]) - 2.0 * dot, 0.0)
                    dx = xi - xev[k]
                    dy = yi - yev[k]
                    dz = zi - zev[k]
                    de = (dx * dx + dy * dy) + dz * dz
                    st = _ins(st, v, de)
                out.append(st)
            return tuple(out)

        init = (big, big, big, big, big, zero, zero, zero, zero, zero)
        sts = lax.fori_loop(0, CPS, chunk_step, (init,) * NSTR)

        merged = sts[0]
        for st_i in range(1, NSTR):
            for slot in range(5):
                merged = _ins(merged, sts[st_i][slot], sts[st_i][5 + slot])

        contrib = zero
        for ek in merged[6:10]:
            d2 = jnp.maximum(ek, EPS)
            ii = lax.bitcast_convert_type(d2, jnp.int32)
            ii = jnp.int32(0x5F3759DF) - lax.shift_right_arithmetic(
                ii, jnp.int32(1))
            y = lax.bitcast_convert_type(ii, jnp.float32)
            y = y * (1.5 - 0.5 * d2 * y * y)
            y = y * (1.5 - 0.5 * d2 * y * y)
            y = y * (1.5 - 0.5 * d2 * y * y)
            dist = d2 * y
            w = jnp.exp(d2 * (-1.0 / H2))
            contrib = contrib + (RADIUS - dist) * w
        return acc + contrib

    acc = lax.fori_loop(0, NG, group_step, zero)
    accv[...] = acc
    pltpu.sync_copy(accv, out_hbm.at[pl.ds(wid * 16, 16)])


def kernel(pred, gt, pcd_radius):
    del pcd_radius
    pred_t = jnp.swapaxes(pred, 1, 2)
    gt_t = jnp.swapaxes(gt, 1, 2)
    pred_tb = lax.optimization_barrier(
        pred_t.astype(jnp.bfloat16)).astype(jnp.float32)

    rep_sc = pl.kernel(
        _rep_body,
        out_type=jax.ShapeDtypeStruct((NTEC * 16,), jnp.float32),
        mesh=plsc.VectorSubcoreMesh(core_axis_name="c", subcore_axis_name="s"),
        scratch_types=[
            pltpu.VMEM((KB * N,), jnp.float32),
            pltpu.VMEM((KB * N,), jnp.float32),
            pltpu.VMEM((KB * N,), jnp.float32),
            pltpu.VMEM((KB * N,), jnp.float32),
            pltpu.VMEM((KB * N,), jnp.float32),
            pltpu.VMEM((KB * N,), jnp.float32),
            pltpu.VMEM((KB * N,), jnp.float32),
            pltpu.VMEM((16,), jnp.float32),
        ],
    )(pred_t.reshape(-1), pred_tb.reshape(-1))

    cd = pl.pallas_call(
        _tc_body,
        grid=(B, NIB),
        in_specs=[
            pl.BlockSpec((1, IB, 3), lambda b, i: (b, i, 0)),
            pl.BlockSpec((1, 3, N), lambda b, i: (b, 0, 0)),
            pl.BlockSpec((1, 3, N), lambda b, i: (b, 0, 0)),
        ],
        out_specs=pl.BlockSpec(memory_space=pltpu.SMEM),
        out_shape=jax.ShapeDtypeStruct((3,), jnp.float32),
        scratch_shapes=[
            pltpu.VMEM((1, N), jnp.float32),
            pltpu.SMEM((3,), jnp.float32),
        ],
    )(pred, pred_t, gt_t)

    cd_loss = (cd[0] + cd[1]) / jnp.float32(B * N) * 100.0
    uniform = (cd[2] + jnp.sum(rep_sc)) / jnp.float32(B * N * 4)
    return (cd_loss, ALPHA * uniform)


# speedup vs baseline: 2.3693x; 1.1598x over previous
import functools

import jax
import jax.numpy as jnp
from jax import lax
from jax.experimental import pallas as pl
from jax.experimental.pallas import tpu as pltpu
from jax.experimental.pallas import tpu_sc as plsc

B = 8
N = 2048
IB = 512
NIB = N // IB
RADIUS = 0.07
H2 = 0.03 * 0.03
EPS = 1e-12
BIG = 1e30
ALPHA = 1.0

BTC = 5
KB = B - BTC
NTEC = 32
QPT = (KB * N) // NTEC
NG = QPT // 16
NSTR = 4
CPS = N // 16 // NSTR


def _tc_body(pred_blk, pred_t, gt_t, out_ref, colmin, accs):
    b = pl.program_id(0)
    ib = pl.program_id(1)

    @pl.when((b == 0) & (ib == 0))
    def _init():
        accs[0] = 0.0
        accs[1] = 0.0
        accs[2] = 0.0

    pi = pred_blk[0]
    pxi = pi[:, 0:1]
    pyi = pi[:, 1:2]
    pzi = pi[:, 2:3]
    a2 = (pxi * pxi + pyi * pyi) + pzi * pzi

    gt = gt_t[0]
    gx = gt[0:1, :]
    gy = gt[1:2, :]
    gz = gt[2:3, :]
    g2 = (gx * gx + gy * gy) + gz * gz

    ab_g = jax.lax.dot_general(
        pi.astype(jnp.bfloat16), gt.astype(jnp.bfloat16),
        (((1,), (0,)), ((), ())),
        preferred_element_type=jnp.float32)
    d_pg = jnp.maximum((a2 + g2) - 2.0 * ab_g, 0.0)

    accs[0] = accs[0] + jnp.sum(jnp.min(d_pg, axis=1))

    col = jnp.min(d_pg, axis=0, keepdims=True)

    @pl.when(ib == 0)
    def _c0():
        colmin[...] = col

    @pl.when(ib > 0)
    def _c1():
        colmin[...] = jnp.minimum(colmin[...], col)

    @pl.when(ib == NIB - 1)
    def _cfin():
        accs[1] = accs[1] + jnp.sum(colmin[...])

    @pl.when(b < BTC)
    def _repulsion():
        pt = pred_t[0]
        px = pt[0:1, :]
        py = pt[1:2, :]
        pz = pt[2:3, :]
        p2 = (px * px + py * py) + pz * pz

        ab_p = jax.lax.dot_general(
            pi.astype(jnp.bfloat16), pt.astype(jnp.bfloat16),
            (((1,), (0,)), ((), ())),
            preferred_element_type=jnp.float32)
        v = jnp.maximum((a2 + p2) - 2.0 * ab_p, 0.0)

        dxp = pxi - px
        dyp = pyi - py
        dzp = pzi - pz
        dex = dxp * dxp + dyp * dyp + dzp * dzp

        jvec = jax.lax.broadcasted_iota(jnp.int32, (IB, N), 1)
        rep = jnp.float32(0.0)
        for r in range(5):
            m = jnp.min(v, axis=1, keepdims=True)
            eq = v == m
            jm = jnp.min(jnp.where(eq, jvec, jnp.int32(N)), axis=1,
                         keepdims=True)
            sel = jvec == jm
            if r > 0:
                dsel = jnp.sum(jnp.where(sel, dex, 0.0), axis=1,
                               keepdims=True)
                d2 = jnp.maximum(dsel, EPS)
                contv = (RADIUS - jnp.sqrt(d2)) * jnp.exp(d2 * (-1.0 / H2))
                rep = rep + jnp.sum(contv)
            if r < 4:
                v = jnp.where(sel, BIG, v)
        accs[2] = accs[2] + rep

    @pl.when((b == B - 1) & (ib == NIB - 1))
    def _out():
        out_ref[0] = accs[0]
        out_ref[1] = accs[1]
        out_ref[2] = accs[2]


def _rep_body(pe_hbm, pb_hbm, out_hbm,
              xe, ye, ze, xb, yb, zb, p2v, accv):
    c = lax.axis_index("c")
    s = lax.axis_index("s")
    wid = s * 2 + c

    for kb in range(KB):
        hb = (BTC + kb) * 3 * N
        lb = kb * N
        pltpu.sync_copy(pe_hbm.at[pl.ds(hb, N)], xe.at[pl.ds(lb, N)])
        pltpu.sync_copy(pe_hbm.at[pl.ds(hb + N, N)], ye.at[pl.ds(lb, N)])
        pltpu.sync_copy(pe_hbm.at[pl.ds(hb + 2 * N, N)], ze.at[pl.ds(lb, N)])
        pltpu.sync_copy(pb_hbm.at[pl.ds(hb, N)], xb.at[pl.ds(lb, N)])
        pltpu.sync_copy(pb_hbm.at[pl.ds(hb + N, N)], yb.at[pl.ds(lb, N)])
        pltpu.sync_copy(pb_hbm.at[pl.ds(hb + 2 * N, N)], zb.at[pl.ds(lb, N)])

    def p2_step(k, carry):
        xv = xe[pl.ds(k * 16, 16)]
        yv = ye[pl.ds(k * 16, 16)]
        zv = ze[pl.ds(k * 16, 16)]
        p2v[pl.ds(k * 16, 16)] = (xv * xv + yv * yv) + zv * zv
        return carry

    lax.fori_loop(0, KB * N // 16, p2_step, jnp.int32(0))

    zero = jnp.zeros((16,), jnp.float32)
    big = jnp.full((16,), BIG, jnp.float32)

    def _ins(st, v, e):
        m1, m2, m3, m4, m5, e1, e2, e3, e4, e5 = st
        c1 = v < m1
        c2 = v < m2
        c3 = v < m3
        c4 = v < m4
        c5 = v < m5
        return (jnp.where(c1, v, m1),
                jnp.where(c1, m1, jnp.where(c2, v, m2)),
                jnp.where(c2, m2, jnp.where(c3, v, m3)),
                jnp.where(c3, m3, jnp.where(c4, v, m4)),
                jnp.where(c4, m4, jnp.where(c5, v, m5)),
                jnp.where(c1, e, e1),
                jnp.where(c1, e1, jnp.where(c2, e, e2)),
                jnp.where(c2, e2, jnp.where(c3, e, e3)),
                jnp.where(c3, e3, jnp.where(c4, e, e4)),
                jnp.where(c4, e4, jnp.where(c5, e, e5)))

    def group_step(g, acc):
        qb = wid * QPT + g * 16
        bi = qb // N
        i0 = bi * N + (qb - bi * N)
        xi = xe[pl.ds(i0, 16)]
        yi = ye[pl.ds(i0, 16)]
        zi = ze[pl.ds(i0, 16)]
        xib = xb[pl.ds(i0, 16)]
        yib = yb[pl.ds(i0, 16)]
        zib = zb[pl.ds(i0, 16)]
        a2i = p2v[pl.ds(i0, 16)]
        jb0 = bi * N

        def chunk_step(ch, sts):
            out = []
            for st_i in range(NSTR):
                st = sts[st_i]
                j0 = jb0 + (st_i * CPS + ch) * 16
                xjv = xb[pl.ds(j0, 16)]
                yjv = yb[pl.ds(j0, 16)]
                zjv = zb[pl.ds(j0, 16)]
                pjv = p2v[pl.ds(j0, 16)]
                xev = xe[pl.ds(j0, 16)]
                yev = ye[pl.ds(j0, 16)]
                zev = ze[pl.ds(j0, 16)]
                for k in range(16):
                    dot = (xib * xjv[k] + yib * yjv[k]) + zib * zjv[k]
                    v = jnp.maximum((a2i + pjv[k]) - 2.0 * dot, 0.0)
                    dx = xi - xev[k]
                    dy = yi - yev[k]
                    dz = zi - zev[k]
                    de = (dx * dx + dy * dy) + dz * dz
                    st = _ins(st, v, de)
                out.append(st)
            return tuple(out)

        init = (big, big, big, big, big, zero, zero, zero, zero, zero)
        sts = lax.fori_loop(0, CPS, chunk_step, (init,) * NSTR)

        merged = sts[0]
        for st_i in range(1, NSTR):
            for slot in range(5):
                merged = _ins(merged, sts[st_i][slot], sts[st_i][5 + slot])

        contrib = zero
        for ek in merged[6:10]:
            d2 = jnp.maximum(ek, EPS)
            ii = lax.bitcast_convert_type(d2, jnp.int32)
            ii = jnp.int32(0x5F3759DF) - lax.shift_right_arithmetic(
                ii, jnp.int32(1))
            y = lax.bitcast_convert_type(ii, jnp.float32)
            y = y * (1.5 - 0.5 * d2 * y * y)
            y = y * (1.5 - 0.5 * d2 * y * y)
            y = y * (1.5 - 0.5 * d2 * y * y)
            dist = d2 * y
            w = jnp.exp(d2 * (-1.0 / H2))
            contrib = contrib + (RADIUS - dist) * w
        return acc + contrib

    acc = lax.fori_loop(0, NG, group_step, zero)
    accv[...] = acc
    pltpu.sync_copy(accv, out_hbm.at[pl.ds(wid * 16, 16)])


def kernel(pred, gt, pcd_radius):
    del pcd_radius
    pred_t = jnp.swapaxes(pred, 1, 2)
    gt_t = jnp.swapaxes(gt, 1, 2)
    pred_tb = lax.optimization_barrier(
        pred_t.astype(jnp.bfloat16)).astype(jnp.float32)

    rep_sc = pl.kernel(
        _rep_body,
        out_type=jax.ShapeDtypeStruct((NTEC * 16,), jnp.float32),
        mesh=plsc.VectorSubcoreMesh(core_axis_name="c", subcore_axis_name="s"),
        scratch_types=[
            pltpu.VMEM((KB * N,), jnp.float32),
            pltpu.VMEM((KB * N,), jnp.float32),
            pltpu.VMEM((KB * N,), jnp.float32),
            pltpu.VMEM((KB * N,), jnp.float32),
            pltpu.VMEM((KB * N,), jnp.float32),
            pltpu.VMEM((KB * N,), jnp.float32),
            pltpu.VMEM((KB * N,), jnp.float32),
            pltpu.VMEM((16,), jnp.float32),
        ],
    )(pred_t.reshape(-1), pred_tb.reshape(-1))

    cd = pl.pallas_call(
        _tc_body,
        grid=(B, NIB),
        in_specs=[
            pl.BlockSpec((1, IB, 3), lambda b, i: (b, i, 0)),
            pl.BlockSpec((1, 3, N), lambda b, i: (b, 0, 0)),
            pl.BlockSpec((1, 3, N), lambda b, i: (b, 0, 0)),
        ],
        out_specs=pl.BlockSpec(memory_space=pltpu.SMEM),
        out_shape=jax.ShapeDtypeStruct((3,), jnp.float32),
        scratch_shapes=[
            pltpu.VMEM((1, N), jnp.float32),
            pltpu.SMEM((3,), jnp.float32),
        ],
    )(pred, pred_t, gt_t)

    cd_loss = (cd[0] + cd[1]) / jnp.float32(B * N) * 100.0
    uniform = (cd[2] + jnp.sum(rep_sc)) / jnp.float32(B * N * 4)
    return (cd_loss, ALPHA * uniform)


# speedup vs baseline: 2.3726x; 1.0014x over previous
import functools

import jax
import jax.numpy as jnp
from jax import lax
from jax.experimental import pallas as pl
from jax.experimental.pallas import tpu as pltpu
from jax.experimental.pallas import tpu_sc as plsc

B = 8
N = 2048
IB = 512
NIB = N // IB
RADIUS = 0.07
H2 = 0.03 * 0.03
EPS = 1e-12
BIG = 1e30
ALPHA = 1.0

BTC = 5
KB = B - BTC
NTEC = 32
QPT = (KB * N) // NTEC
NG = QPT // 16
NSTR = 4
CPS = N // 16 // NSTR


def _tc_body(pred_blk, pred_t, gt_t, out_ref, colmin, accs):
    b = pl.program_id(0)
    ib = pl.program_id(1)

    @pl.when((b == 0) & (ib == 0))
    def _init():
        accs[0] = 0.0
        accs[1] = 0.0
        accs[2] = 0.0

    pi = pred_blk[0]
    pxi = pi[:, 0:1]
    pyi = pi[:, 1:2]
    pzi = pi[:, 2:3]
    a2 = (pxi * pxi + pyi * pyi) + pzi * pzi

    gt = gt_t[0]
    gx = gt[0:1, :]
    gy = gt[1:2, :]
    gz = gt[2:3, :]
    g2 = (gx * gx + gy * gy) + gz * gz

    ab_g = jax.lax.dot_general(
        pi.astype(jnp.bfloat16), gt.astype(jnp.bfloat16),
        (((1,), (0,)), ((), ())),
        preferred_element_type=jnp.float32)
    d_pg = jnp.maximum((a2 + g2) - 2.0 * ab_g, 0.0)

    accs[0] = accs[0] + jnp.sum(jnp.min(d_pg, axis=1))

    col = jnp.min(d_pg, axis=0, keepdims=True)

    @pl.when(ib == 0)
    def _c0():
        colmin[...] = col

    @pl.when(ib > 0)
    def _c1():
        colmin[...] = jnp.minimum(colmin[...], col)

    @pl.when(ib == NIB - 1)
    def _cfin():
        accs[1] = accs[1] + jnp.sum(colmin[...])

    @pl.when(b < BTC)
    def _repulsion():
        pt = pred_t[0]
        px = pt[0:1, :]
        py = pt[1:2, :]
        pz = pt[2:3, :]
        p2 = (px * px + py * py) + pz * pz

        ab_p = jax.lax.dot_general(
            pi.astype(jnp.bfloat16), pt.astype(jnp.bfloat16),
            (((1,), (0,)), ((), ())),
            preferred_element_type=jnp.float32)
        v = jnp.maximum((a2 + p2) - 2.0 * ab_p, 0.0)

        dxp = pxi - px
        dyp = pyi - py
        dzp = pzi - pz
        dex = dxp * dxp + dyp * dyp + dzp * dzp

        jvec = jax.lax.broadcasted_iota(jnp.int32, (IB, N), 1).astype(jnp.float32)
        rep = jnp.float32(0.0)
        for r in range(5):
            m = jnp.min(v, axis=1, keepdims=True)
            eq = v == m
            jm = jnp.min(jnp.where(eq, jvec, jnp.float32(N)), axis=1,
                         keepdims=True)
            sel = jvec == jm
            if r > 0:
                dsel = jnp.sum(jnp.where(sel, dex, 0.0), axis=1,
                               keepdims=True)
                d2 = jnp.maximum(dsel, EPS)
                contv = (RADIUS - jnp.sqrt(d2)) * jnp.exp(d2 * (-1.0 / H2))
                rep = rep + jnp.sum(contv)
            if r < 4:
                v = jnp.where(sel, BIG, v)
        accs[2] = accs[2] + rep

    @pl.when((b == B - 1) & (ib == NIB - 1))
    def _out():
        out_ref[0] = accs[0]
        out_ref[1] = accs[1]
        out_ref[2] = accs[2]


def _rep_body(pe_hbm, pb_hbm, out_hbm,
              xe, ye, ze, xb, yb, zb, p2v, accv):
    c = lax.axis_index("c")
    s = lax.axis_index("s")
    wid = s * 2 + c

    for kb in range(KB):
        hb = (BTC + kb) * 3 * N
        lb = kb * N
        pltpu.sync_copy(pe_hbm.at[pl.ds(hb, N)], xe.at[pl.ds(lb, N)])
        pltpu.sync_copy(pe_hbm.at[pl.ds(hb + N, N)], ye.at[pl.ds(lb, N)])
        pltpu.sync_copy(pe_hbm.at[pl.ds(hb + 2 * N, N)], ze.at[pl.ds(lb, N)])
        pltpu.sync_copy(pb_hbm.at[pl.ds(hb, N)], xb.at[pl.ds(lb, N)])
        pltpu.sync_copy(pb_hbm.at[pl.ds(hb + N, N)], yb.at[pl.ds(lb, N)])
        pltpu.sync_copy(pb_hbm.at[pl.ds(hb + 2 * N, N)], zb.at[pl.ds(lb, N)])

    def p2_step(k, carry):
        xv = xe[pl.ds(k * 16, 16)]
        yv = ye[pl.ds(k * 16, 16)]
        zv = ze[pl.ds(k * 16, 16)]
        p2v[pl.ds(k * 16, 16)] = (xv * xv + yv * yv) + zv * zv
        return carry

    lax.fori_loop(0, KB * N // 16, p2_step, jnp.int32(0))

    zero = jnp.zeros((16,), jnp.float32)
    big = jnp.full((16,), BIG, jnp.float32)

    def _ins(st, v, e):
        m1, m2, m3, m4, m5, e1, e2, e3, e4, e5 = st
        c1 = v < m1
        c2 = v < m2
        c3 = v < m3
        c4 = v < m4
        c5 = v < m5
        return (jnp.where(c1, v, m1),
                jnp.where(c1, m1, jnp.where(c2, v, m2)),
                jnp.where(c2, m2, jnp.where(c3, v, m3)),
                jnp.where(c3, m3, jnp.where(c4, v, m4)),
                jnp.where(c4, m4, jnp.where(c5, v, m5)),
                jnp.where(c1, e, e1),
                jnp.where(c1, e1, jnp.where(c2, e, e2)),
                jnp.where(c2, e2, jnp.where(c3, e, e3)),
                jnp.where(c3, e3, jnp.where(c4, e, e4)),
                jnp.where(c4, e4, jnp.where(c5, e, e5)))

    def group_step(g, acc):
        qb = wid * QPT + g * 16
        bi = qb // N
        i0 = bi * N + (qb - bi * N)
        xi = xe[pl.ds(i0, 16)]
        yi = ye[pl.ds(i0, 16)]
        zi = ze[pl.ds(i0, 16)]
        xib = xb[pl.ds(i0, 16)]
        yib = yb[pl.ds(i0, 16)]
        zib = zb[pl.ds(i0, 16)]
        a2i = p2v[pl.ds(i0, 16)]
        jb0 = bi * N

        def chunk_step(ch, sts):
            out = []
            for st_i in range(NSTR):
                st = sts[st_i]
                j0 = jb0 + (st_i * CPS + ch) * 16
                xjv = xb[pl.ds(j0, 16)]
                yjv = yb[pl.ds(j0, 16)]
                zjv = zb[pl.ds(j0, 16)]
                pjv = p2v[pl.ds(j0, 16)]
                xev = xe[pl.ds(j0, 16)]
                yev = ye[pl.ds(j0, 16)]
                zev = ze[pl.ds(j0, 16)]
                for k in range(16):
                    dot = (xib * xjv[k] + yib * yjv[k]) + zib * zjv[k]
                    v = jnp.maximum((a2i + pjv[k]) - 2.0 * dot, 0.0)
                    dx = xi - xev[k]
                    dy = yi - yev[k]
                    dz = zi - zev[k]
                    de = (dx * dx + dy * dy) + dz * dz
                    st = _ins(st, v, de)
                out.append(st)
            return tuple(out)

        init = (big, big, big, big, big, zero, zero, zero, zero, zero)
        sts = lax.fori_loop(0, CPS, chunk_step, (init,) * NSTR)

        merged = sts[0]
        for st_i in range(1, NSTR):
            for slot in range(5):
                merged = _ins(merged, sts[st_i][slot], sts[st_i][5 + slot])

        contrib = zero
        for ek in merged[6:10]:
            d2 = jnp.maximum(ek, EPS)
            ii = lax.bitcast_convert_type(d2, jnp.int32)
            ii = jnp.int32(0x5F3759DF) - lax.shift_right_arithmetic(
                ii, jnp.int32(1))
            y = lax.bitcast_convert_type(ii, jnp.float32)
            y = y * (1.5 - 0.5 * d2 * y * y)
            y = y * (1.5 - 0.5 * d2 * y * y)
            y = y * (1.5 - 0.5 * d2 * y * y)
            dist = d2 * y
            w = jnp.exp(d2 * (-1.0 / H2))
            contrib = contrib + (RADIUS - dist) * w
        return acc + contrib

    acc = lax.fori_loop(0, NG, group_step, zero)
    accv[...] = acc
    pltpu.sync_copy(accv, out_hbm.at[pl.ds(wid * 16, 16)])


def kernel(pred, gt, pcd_radius):
    del pcd_radius
    pred_t = jnp.swapaxes(pred, 1, 2)
    gt_t = jnp.swapaxes(gt, 1, 2)
    pred_tb = lax.optimization_barrier(
        pred_t.astype(jnp.bfloat16)).astype(jnp.float32)

    rep_sc = pl.kernel(
        _rep_body,
        out_type=jax.ShapeDtypeStruct((NTEC * 16,), jnp.float32),
        mesh=plsc.VectorSubcoreMesh(core_axis_name="c", subcore_axis_name="s"),
        scratch_types=[
            pltpu.VMEM((KB * N,), jnp.float32),
            pltpu.VMEM((KB * N,), jnp.float32),
            pltpu.VMEM((KB * N,), jnp.float32),
            pltpu.VMEM((KB * N,), jnp.float32),
            pltpu.VMEM((KB * N,), jnp.float32),
            pltpu.VMEM((KB * N,), jnp.float32),
            pltpu.VMEM((KB * N,), jnp.float32),
            pltpu.VMEM((16,), jnp.float32),
        ],
    )(pred_t.reshape(-1), pred_tb.reshape(-1))

    cd = pl.pallas_call(
        _tc_body,
        grid=(B, NIB),
        in_specs=[
            pl.BlockSpec((1, IB, 3), lambda b, i: (b, i, 0)),
            pl.BlockSpec((1, 3, N), lambda b, i: (b, 0, 0)),
            pl.BlockSpec((1, 3, N), lambda b, i: (b, 0, 0)),
        ],
        out_specs=pl.BlockSpec(memory_space=pltpu.SMEM),
        out_shape=jax.ShapeDtypeStruct((3,), jnp.float32),
        scratch_shapes=[
            pltpu.VMEM((1, N), jnp.float32),
            pltpu.SMEM((3,), jnp.float32),
        ],
    )(pred, pred_t, gt_t)

    cd_loss = (cd[0] + cd[1]) / jnp.float32(B * N) * 100.0
    uniform = (cd[2] + jnp.sum(rep_sc)) / jnp.float32(B * N * 4)
    return (cd_loss, ALPHA * uniform)


# speedup vs baseline: 2.4340x; 1.0259x over previous
import functools

import jax
import jax.numpy as jnp
from jax import lax
from jax.experimental import pallas as pl
from jax.experimental.pallas import tpu as pltpu
from jax.experimental.pallas import tpu_sc as plsc

B = 8
N = 2048
IB = 512
NIB = N // IB
RADIUS = 0.07
H2 = 0.03 * 0.03
EPS = 1e-12
BIG = 1e30
ALPHA = 1.0

BTC = 6
KB = B - BTC
NTEC = 32
QPT = (KB * N) // NTEC
NG = QPT // 16
NSTR = 4
CPS = N // 16 // NSTR


def _tc_body(pred_blk, pred_t, gt_t, out_ref, colmin, accs):
    b = pl.program_id(0)
    ib = pl.program_id(1)

    @pl.when((b == 0) & (ib == 0))
    def _init():
        accs[0] = 0.0
        accs[1] = 0.0
        accs[2] = 0.0

    pi = pred_blk[0]
    pxi = pi[:, 0:1]
    pyi = pi[:, 1:2]
    pzi = pi[:, 2:3]
    a2 = (pxi * pxi + pyi * pyi) + pzi * pzi

    gt = gt_t[0]
    gx = gt[0:1, :]
    gy = gt[1:2, :]
    gz = gt[2:3, :]
    g2 = (gx * gx + gy * gy) + gz * gz

    ab_g = jax.lax.dot_general(
        pi.astype(jnp.bfloat16), gt.astype(jnp.bfloat16),
        (((1,), (0,)), ((), ())),
        preferred_element_type=jnp.float32)
    d_pg = jnp.maximum((a2 + g2) - 2.0 * ab_g, 0.0)

    accs[0] = accs[0] + jnp.sum(jnp.min(d_pg, axis=1))

    col = jnp.min(d_pg, axis=0, keepdims=True)

    @pl.when(ib == 0)
    def _c0():
        colmin[...] = col

    @pl.when(ib > 0)
    def _c1():
        colmin[...] = jnp.minimum(colmin[...], col)

    @pl.when(ib == NIB - 1)
    def _cfin():
        accs[1] = accs[1] + jnp.sum(colmin[...])

    @pl.when(b < BTC)
    def _repulsion():
        pt = pred_t[0]
        px = pt[0:1, :]
        py = pt[1:2, :]
        pz = pt[2:3, :]
        p2 = (px * px + py * py) + pz * pz

        ab_p = jax.lax.dot_general(
            pi.astype(jnp.bfloat16), pt.astype(jnp.bfloat16),
            (((1,), (0,)), ((), ())),
            preferred_element_type=jnp.float32)
        v = jnp.maximum((a2 + p2) - 2.0 * ab_p, 0.0)

        dxp = pxi - px
        dyp = pyi - py
        dzp = pzi - pz
        dex = dxp * dxp + dyp * dyp + dzp * dzp

        jvec = jax.lax.broadcasted_iota(jnp.int32, (IB, N), 1).astype(jnp.float32)
        rep = jnp.float32(0.0)
        for r in range(5):
            m = jnp.min(v, axis=1, keepdims=True)
            eq = v == m
            jm = jnp.min(jnp.where(eq, jvec, jnp.float32(N)), axis=1,
                         keepdims=True)
            sel = jvec == jm
            if r > 0:
                dsel = jnp.sum(jnp.where(sel, dex, 0.0), axis=1,
                               keepdims=True)
                d2 = jnp.maximum(dsel, EPS)
                contv = (RADIUS - jnp.sqrt(d2)) * jnp.exp(d2 * (-1.0 / H2))
                rep = rep + jnp.sum(contv)
            if r < 4:
                v = jnp.where(sel, BIG, v)
        accs[2] = accs[2] + rep

    @pl.when((b == B - 1) & (ib == NIB - 1))
    def _out():
        out_ref[0] = accs[0]
        out_ref[1] = accs[1]
        out_ref[2] = accs[2]


def _rep_body(pe_hbm, pb_hbm, out_hbm,
              xe, ye, ze, xb, yb, zb, p2v, accv):
    c = lax.axis_index("c")
    s = lax.axis_index("s")
    wid = s * 2 + c

    for kb in range(KB):
        hb = (BTC + kb) * 3 * N
        lb = kb * N
        pltpu.sync_copy(pe_hbm.at[pl.ds(hb, N)], xe.at[pl.ds(lb, N)])
        pltpu.sync_copy(pe_hbm.at[pl.ds(hb + N, N)], ye.at[pl.ds(lb, N)])
        pltpu.sync_copy(pe_hbm.at[pl.ds(hb + 2 * N, N)], ze.at[pl.ds(lb, N)])
        pltpu.sync_copy(pb_hbm.at[pl.ds(hb, N)], xb.at[pl.ds(lb, N)])
        pltpu.sync_copy(pb_hbm.at[pl.ds(hb + N, N)], yb.at[pl.ds(lb, N)])
        pltpu.sync_copy(pb_hbm.at[pl.ds(hb + 2 * N, N)], zb.at[pl.ds(lb, N)])

    def p2_step(k, carry):
        xv = xe[pl.ds(k * 16, 16)]
        yv = ye[pl.ds(k * 16, 16)]
        zv = ze[pl.ds(k * 16, 16)]
        p2v[pl.ds(k * 16, 16)] = (xv * xv + yv * yv) + zv * zv
        return carry

    lax.fori_loop(0, KB * N // 16, p2_step, jnp.int32(0))

    zero = jnp.zeros((16,), jnp.float32)
    big = jnp.full((16,), BIG, jnp.float32)

    def _ins(st, v, e):
        m1, m2, m3, m4, m5, e1, e2, e3, e4, e5 = st
        c1 = v < m1
        c2 = v < m2
        c3 = v < m3
        c4 = v < m4
        c5 = v < m5
        return (jnp.where(c1, v, m1),
                jnp.where(c1, m1, jnp.where(c2, v, m2)),
                jnp.where(c2, m2, jnp.where(c3, v, m3)),
                jnp.where(c3, m3, jnp.where(c4, v, m4)),
                jnp.where(c4, m4, jnp.where(c5, v, m5)),
                jnp.where(c1, e, e1),
                jnp.where(c1, e1, jnp.where(c2, e, e2)),
                jnp.where(c2, e2, jnp.where(c3, e, e3)),
                jnp.where(c3, e3, jnp.where(c4, e, e4)),
                jnp.where(c4, e4, jnp.where(c5, e, e5)))

    def group_step(g, acc):
        qb = wid * QPT + g * 16
        bi = qb // N
        i0 = bi * N + (qb - bi * N)
        xi = xe[pl.ds(i0, 16)]
        yi = ye[pl.ds(i0, 16)]
        zi = ze[pl.ds(i0, 16)]
        xib = xb[pl.ds(i0, 16)]
        yib = yb[pl.ds(i0, 16)]
        zib = zb[pl.ds(i0, 16)]
        a2i = p2v[pl.ds(i0, 16)]
        jb0 = bi * N

        def chunk_step(ch, sts):
            out = []
            for st_i in range(NSTR):
                st = sts[st_i]
                j0 = jb0 + (st_i * CPS + ch) * 16
                xjv = xb[pl.ds(j0, 16)]
                yjv = yb[pl.ds(j0, 16)]
                zjv = zb[pl.ds(j0, 16)]
                pjv = p2v[pl.ds(j0, 16)]
                xev = xe[pl.ds(j0, 16)]
                yev = ye[pl.ds(j0, 16)]
                zev = ze[pl.ds(j0, 16)]
                for k in range(16):
                    dot = (xib * xjv[k] + yib * yjv[k]) + zib * zjv[k]
                    v = jnp.maximum((a2i + pjv[k]) - 2.0 * dot, 0.0)
                    dx = xi - xev[k]
                    dy = yi - yev[k]
                    dz = zi - zev[k]
                    de = (dx * dx + dy * dy) + dz * dz
                    st = _ins(st, v, de)
                out.append(st)
            return tuple(out)

        init = (big, big, big, big, big, zero, zero, zero, zero, zero)
        sts = lax.fori_loop(0, CPS, chunk_step, (init,) * NSTR)

        merged = sts[0]
        for st_i in range(1, NSTR):
            for slot in range(5):
                merged = _ins(merged, sts[st_i][slot], sts[st_i][5 + slot])

        contrib = zero
        for ek in merged[6:10]:
            d2 = jnp.maximum(ek, EPS)
            ii = lax.bitcast_convert_type(d2, jnp.int32)
            ii = jnp.int32(0x5F3759DF) - lax.shift_right_arithmetic(
                ii, jnp.int32(1))
            y = lax.bitcast_convert_type(ii, jnp.float32)
            y = y * (1.5 - 0.5 * d2 * y * y)
            y = y * (1.5 - 0.5 * d2 * y * y)
            y = y * (1.5 - 0.5 * d2 * y * y)
            dist = d2 * y
            w = jnp.exp(d2 * (-1.0 / H2))
            contrib = contrib + (RADIUS - dist) * w
        return acc + contrib

    acc = lax.fori_loop(0, NG, group_step, zero)
    accv[...] = acc
    pltpu.sync_copy(accv, out_hbm.at[pl.ds(wid * 16, 16)])


def kernel(pred, gt, pcd_radius):
    del pcd_radius
    pred_t = jnp.swapaxes(pred, 1, 2)
    gt_t = jnp.swapaxes(gt, 1, 2)
    pred_tb = lax.optimization_barrier(
        pred_t.astype(jnp.bfloat16)).astype(jnp.float32)

    rep_sc = pl.kernel(
        _rep_body,
        out_type=jax.ShapeDtypeStruct((NTEC * 16,), jnp.float32),
        mesh=plsc.VectorSubcoreMesh(core_axis_name="c", subcore_axis_name="s"),
        scratch_types=[
            pltpu.VMEM((KB * N,), jnp.float32),
            pltpu.VMEM((KB * N,), jnp.float32),
            pltpu.VMEM((KB * N,), jnp.float32),
            pltpu.VMEM((KB * N,), jnp.float32),
            pltpu.VMEM((KB * N,), jnp.float32),
            pltpu.VMEM((KB * N,), jnp.float32),
            pltpu.VMEM((KB * N,), jnp.float32),
            pltpu.VMEM((16,), jnp.float32),
        ],
    )(pred_t.reshape(-1), pred_tb.reshape(-1))

    cd = pl.pallas_call(
        _tc_body,
        grid=(B, NIB),
        in_specs=[
            pl.BlockSpec((1, IB, 3), lambda b, i: (b, i, 0)),
            pl.BlockSpec((1, 3, N), lambda b, i: (b, 0, 0)),
            pl.BlockSpec((1, 3, N), lambda b, i: (b, 0, 0)),
        ],
        out_specs=pl.BlockSpec(memory_space=pltpu.SMEM),
        out_shape=jax.ShapeDtypeStruct((3,), jnp.float32),
        scratch_shapes=[
            pltpu.VMEM((1, N), jnp.float32),
            pltpu.SMEM((3,), jnp.float32),
        ],
    )(pred, pred_t, gt_t)

    cd_loss = (cd[0] + cd[1]) / jnp.float32(B * N) * 100.0
    uniform = (cd[2] + jnp.sum(rep_sc)) / jnp.float32(B * N * 4)
    return (cd_loss, ALPHA * uniform)


# speedup vs baseline: 2.4376x; 1.0015x over previous
import jax
import jax.numpy as jnp
from jax import lax
from jax.experimental import pallas as pl
from jax.experimental.pallas import tpu as pltpu
from jax.experimental.pallas import tpu_sc as plsc

B = 8
N = 2048
IB = 512
NIB = N // IB
RADIUS = 0.07
H2 = 0.03 * 0.03
EPS = 1e-12
BIG = 1e30
ALPHA = 1.0

BTC = 6
KB = B - BTC
NTEC = 32
QPT = (KB * N) // NTEC
NG = QPT // 16
NSTR = 4
CPS = N // 16 // NSTR


def _tc_body(pred_blk, pred_t, gt_t, out_ref, colmin, accs):
    b = pl.program_id(0)
    ib = pl.program_id(1)

    @pl.when((b == 0) & (ib == 0))
    def _init():
        accs[0] = 0.0
        accs[1] = 0.0
        accs[2] = 0.0

    pi = pred_blk[0]
    pxi = pi[:, 0:1]
    pyi = pi[:, 1:2]
    pzi = pi[:, 2:3]
    a2 = (pxi * pxi + pyi * pyi) + pzi * pzi

    gt = gt_t[0]
    gx = gt[0:1, :]
    gy = gt[1:2, :]
    gz = gt[2:3, :]
    g2 = (gx * gx + gy * gy) + gz * gz

    ab_g = jax.lax.dot_general(
        pi.astype(jnp.bfloat16), gt.astype(jnp.bfloat16),
        (((1,), (0,)), ((), ())),
        preferred_element_type=jnp.float32)
    d_pg = jnp.maximum((a2 + g2) - 2.0 * ab_g, 0.0)

    accs[0] = accs[0] + jnp.sum(jnp.min(d_pg, axis=1))

    col = jnp.min(d_pg, axis=0, keepdims=True)

    @pl.when(ib == 0)
    def _c0():
        colmin[...] = col

    @pl.when(ib > 0)
    def _c1():
        colmin[...] = jnp.minimum(colmin[...], col)

    @pl.when(ib == NIB - 1)
    def _cfin():
        accs[1] = accs[1] + jnp.sum(colmin[...])

    @pl.when(b < BTC)
    def _repulsion():
        pt = pred_t[0]
        px = pt[0:1, :]
        py = pt[1:2, :]
        pz = pt[2:3, :]
        p2 = (px * px + py * py) + pz * pz

        ab_p = jax.lax.dot_general(
            pi.astype(jnp.bfloat16), pt.astype(jnp.bfloat16),
            (((1,), (0,)), ((), ())),
            preferred_element_type=jnp.float32)
        v = jnp.maximum((a2 + p2) - 2.0 * ab_p, 0.0)

        dxp = pxi - px
        dyp = pyi - py
        dzp = pzi - pz
        dex = dxp * dxp + dyp * dyp + dzp * dzp

        jvec = jax.lax.broadcasted_iota(jnp.int32, (IB, N), 1).astype(jnp.float32)
        rep = jnp.float32(0.0)
        for r in range(5):
            m = jnp.min(v, axis=1, keepdims=True)
            eq = v == m
            jm = jnp.min(jnp.where(eq, jvec, jnp.float32(N)), axis=1,
                         keepdims=True)
            sel = jvec == jm
            if r > 0:
                dsel = jnp.sum(jnp.where(sel, dex, 0.0), axis=1,
                               keepdims=True)
                d2 = jnp.maximum(dsel, EPS)
                contv = (RADIUS - jnp.sqrt(d2)) * jnp.exp(d2 * (-1.0 / H2))
                rep = rep + jnp.sum(contv)
            if r < 4:
                v = jnp.where(sel, BIG, v)
        accs[2] = accs[2] + rep

    @pl.when((b == B - 1) & (ib == NIB - 1))
    def _out():
        out_ref[0] = accs[0]
        out_ref[1] = accs[1]
        out_ref[2] = accs[2]


def _rep_body(pe_hbm, pb_hbm, out_hbm,
              xe, ye, ze, xb, yb, zb, p2v, accv):
    c = lax.axis_index("c")
    s = lax.axis_index("s")
    wid = s * 2 + c

    for kb in range(KB):
        hb = (BTC + kb) * 3 * N
        lb = kb * N
        pltpu.sync_copy(pe_hbm.at[pl.ds(hb, N)], xe.at[pl.ds(lb, N)])
        pltpu.sync_copy(pe_hbm.at[pl.ds(hb + N, N)], ye.at[pl.ds(lb, N)])
        pltpu.sync_copy(pe_hbm.at[pl.ds(hb + 2 * N, N)], ze.at[pl.ds(lb, N)])
        pltpu.sync_copy(pb_hbm.at[pl.ds(hb, N)], xb.at[pl.ds(lb, N)])
        pltpu.sync_copy(pb_hbm.at[pl.ds(hb + N, N)], yb.at[pl.ds(lb, N)])
        pltpu.sync_copy(pb_hbm.at[pl.ds(hb + 2 * N, N)], zb.at[pl.ds(lb, N)])

    def p2_step(k, carry):
        xv = xe[pl.ds(k * 16, 16)]
        yv = ye[pl.ds(k * 16, 16)]
        zv = ze[pl.ds(k * 16, 16)]
        p2v[pl.ds(k * 16, 16)] = (xv * xv + yv * yv) + zv * zv
        return carry

    lax.fori_loop(0, KB * N // 16, p2_step, jnp.int32(0))

    zero = jnp.zeros((16,), jnp.float32)
    big = jnp.full((16,), BIG, jnp.float32)

    def _ins(st, v, e):
        m1, m2, m3, m4, m5, e1, e2, e3, e4, e5 = st
        c1 = v < m1
        c2 = v < m2
        c3 = v < m3
        c4 = v < m4
        c5 = v < m5
        return (jnp.where(c1, v, m1),
                jnp.where(c1, m1, jnp.where(c2, v, m2)),
                jnp.where(c2, m2, jnp.where(c3, v, m3)),
                jnp.where(c3, m3, jnp.where(c4, v, m4)),
                jnp.where(c4, m4, jnp.where(c5, v, m5)),
                jnp.where(c1, e, e1),
                jnp.where(c1, e1, jnp.where(c2, e, e2)),
                jnp.where(c2, e2, jnp.where(c3, e, e3)),
                jnp.where(c3, e3, jnp.where(c4, e, e4)),
                jnp.where(c4, e4, jnp.where(c5, e, e5)))

    def group_step(g, acc):
        qb = wid * QPT + g * 16
        bi = qb // N
        i0 = bi * N + (qb - bi * N)
        xi = xe[pl.ds(i0, 16)]
        yi = ye[pl.ds(i0, 16)]
        zi = ze[pl.ds(i0, 16)]
        xib = xb[pl.ds(i0, 16)]
        yib = yb[pl.ds(i0, 16)]
        zib = zb[pl.ds(i0, 16)]
        a2i = p2v[pl.ds(i0, 16)]
        jb0 = bi * N

        def chunk_step(ch, sts):
            out = []
            for st_i in range(NSTR):
                st = sts[st_i]
                j0 = jb0 + (st_i * CPS + ch) * 16
                xjv = xb[pl.ds(j0, 16)]
                yjv = yb[pl.ds(j0, 16)]
                zjv = zb[pl.ds(j0, 16)]
                pjv = p2v[pl.ds(j0, 16)]
                xev = xe[pl.ds(j0, 16)]
                yev = ye[pl.ds(j0, 16)]
                zev = ze[pl.ds(j0, 16)]
                for k in range(16):
                    dot = (xib * xjv[k] + yib * yjv[k]) + zib * zjv[k]
                    v = jnp.maximum((a2i + pjv[k]) - 2.0 * dot, 0.0)
                    dx = xi - xev[k]
                    dy = yi - yev[k]
                    dz = zi - zev[k]
                    de = (dx * dx + dy * dy) + dz * dz
                    st = _ins(st, v, de)
                out.append(st)
            return tuple(out)

        init = (big, big, big, big, big, zero, zero, zero, zero, zero)
        sts = lax.fori_loop(0, CPS, chunk_step, (init,) * NSTR)

        merged = sts[0]
        for st_i in range(1, NSTR):
            for slot in range(5):
                merged = _ins(merged, sts[st_i][slot], sts[st_i][5 + slot])

        contrib = zero
        for ek in merged[6:10]:
            d2 = jnp.maximum(ek, EPS)
            ii = lax.bitcast_convert_type(d2, jnp.int32)
            ii = jnp.int32(0x5F3759DF) - lax.shift_right_arithmetic(
                ii, jnp.int32(1))
            y = lax.bitcast_convert_type(ii, jnp.float32)
            y = y * (1.5 - 0.5 * d2 * y * y)
            y = y * (1.5 - 0.5 * d2 * y * y)
            y = y * (1.5 - 0.5 * d2 * y * y)
            dist = d2 * y
            w = jnp.exp(d2 * (-1.0 / H2))
            contrib = contrib + (RADIUS - dist) * w
        return acc + contrib

    acc = lax.fori_loop(0, NG, group_step, zero)
    accv[...] = acc
    pltpu.sync_copy(accv, out_hbm.at[pl.ds(wid * 16, 16)])


def kernel(pred, gt, pcd_radius):
    del pcd_radius
    pred_t = jnp.swapaxes(pred, 1, 2)
    gt_t = jnp.swapaxes(gt, 1, 2)
    pred_tb = lax.optimization_barrier(
        pred_t.astype(jnp.bfloat16)).astype(jnp.float32)

    rep_sc = pl.kernel(
        _rep_body,
        out_type=jax.ShapeDtypeStruct((NTEC * 16,), jnp.float32),
        mesh=plsc.VectorSubcoreMesh(core_axis_name="c", subcore_axis_name="s"),
        scratch_types=[
            pltpu.VMEM((KB * N,), jnp.float32),
            pltpu.VMEM((KB * N,), jnp.float32),
            pltpu.VMEM((KB * N,), jnp.float32),
            pltpu.VMEM((KB * N,), jnp.float32),
            pltpu.VMEM((KB * N,), jnp.float32),
            pltpu.VMEM((KB * N,), jnp.float32),
            pltpu.VMEM((KB * N,), jnp.float32),
            pltpu.VMEM((16,), jnp.float32),
        ],
    )(pred_t.reshape(-1), pred_tb.reshape(-1))

    cd = pl.pallas_call(
        _tc_body,
        grid=(B, NIB),
        in_specs=[
            pl.BlockSpec((1, IB, 3), lambda b, i: (b, i, 0)),
            pl.BlockSpec((1, 3, N), lambda b, i: (b, 0, 0)),
            pl.BlockSpec((1, 3, N), lambda b, i: (b, 0, 0)),
        ],
        out_specs=pl.BlockSpec(memory_space=pltpu.SMEM),
        out_shape=jax.ShapeDtypeStruct((3,), jnp.float32),
        scratch_shapes=[
            pltpu.VMEM((1, N), jnp.float32),
            pltpu.SMEM((3,), jnp.float32),
        ],
    )(pred, pred_t, gt_t)

    cd_loss = (cd[0] + cd[1]) / jnp.float32(B * N) * 100.0
    uniform = (cd[2] + jnp.sum(rep_sc)) / jnp.float32(B * N * 4)
    return (cd_loss, ALPHA * uniform)
